# baseline (speedup 1.0000x reference)
import functools

import jax
import jax.numpy as jnp
from jax import lax
from jax.experimental import pallas as pl
from jax.experimental.pallas import tpu as pltpu
from jax.experimental.pallas import tpu_sc as plsc

NC = 2
NS = 16
NW = NC * NS
L = 16


@functools.lru_cache(maxsize=None)
def _make_kernel(B, S, V, D, C):
    s_per_w = S // NW
    chunks = s_per_w // C
    ncol = D // L
    npairs = chunks * 2

    mesh = plsc.VectorSubcoreMesh(core_axis_name="c", subcore_axis_name="s")

    @functools.partial(
        pl.kernel,
        mesh=mesh,
        out_type=jax.ShapeDtypeStruct((B * S, D), jnp.float32),
        scratch_types=[
            pltpu.VMEM((B, s_per_w), jnp.int32),
            pltpu.VMEM((8, C, D), jnp.float32),
            pltpu.VMEM((2, C, D), jnp.float32),
            pltpu.SemaphoreType.DMA((8,)),
            pltpu.SemaphoreType.DMA((8,)),
            pltpu.SemaphoreType.DMA((2,)),
        ],
    )
    def emb_kernel(ids_hbm, tok_hbm, pos_hbm, out_hbm, idx_v, tokb, posb, gsem, ssem, psem):
        wid = lax.axis_index("s") * NC + lax.axis_index("c")
        s0 = wid * s_per_w

        for b in range(B):
            pltpu.sync_copy(ids_hbm.at[pl.ds(b * S + s0, s_per_w)], idx_v.at[b])

        def gather(k, g, h, p):
            b = h * 2 + p
            pltpu.async_copy(
                tok_hbm.at[idx_v.at[b, pl.ds(k * C, C)]],
                tokb.at[g * 2 + p],
                gsem.at[g * 2 + p],
            )

        def gather_wait(g, p):
            pltpu.make_async_copy(
                tok_hbm.at[pl.ds(0, C)], tokb.at[g * 2 + p], gsem.at[g * 2 + p]
            ).wait()

        def scatter(k, g, h, p):
            b = h * 2 + p
            pltpu.async_copy(
                tokb.at[g * 2 + p],
                out_hbm.at[pl.ds(b * S + s0 + k * C, C)],
                ssem.at[g * 2 + p],
            )

        def scatter_wait(g, p):
            pltpu.make_async_copy(
                tokb.at[g * 2 + p], out_hbm.at[pl.ds(0, C)], ssem.at[g * 2 + p]
            ).wait()

        def pos_load(k, pb):
            pltpu.async_copy(pos_hbm.at[pl.ds(s0 + k * C, C)], posb.at[pb], psem.at[pb])

        def pos_wait(pb):
            pltpu.make_async_copy(
                pos_hbm.at[pl.ds(s0, C)], posb.at[pb], psem.at[pb]
            ).wait()

        pos_load(0, 0)
        for p in range(2):
            gather(0, 0, 0, p)
        for p in range(2):
            gather(0, 1, 1, p)

        def outer(j, carry):
            for tt in range(4):
                k = 2 * j + tt // 2
                h = tt % 2
                g = tt
                pb = tt // 2
                g2 = (tt + 2) % 4

                if h == 0:
                    pos_wait(pb)
                    if tt == 0:
                        pos_load(k + 1, 1 - pb)
                    else:
                        @pl.when(j < chunks // 2 - 1)
                        def _():
                            pos_load(k + 1, 1 - pb)

                for p in range(2):
                    gather_wait(g, p)

                if tt < 2:
                    @pl.when(j >= 1)
                    def _():
                        for p in range(2):
                            scatter_wait(g2, p)
                    for p in range(2):
                        gather(k + 1, g2, h, p)
                else:
                    @pl.when(j < chunks // 2 - 1)
                    def _():
                        for p in range(2):
                            scatter_wait(g2, p)
                            gather(k + 1, g2, h, p)

                def row_body(r, c2):
                    for c in range(ncol):
                        sl = pl.ds(c * L, L)
                        pv = posb[pb, r, sl]
                        for p in range(2):
                            tokb[g * 2 + p, r, sl] = tokb[g * 2 + p, r, sl] + pv
                    return c2

                lax.fori_loop(0, C, row_body, 0)

                for p in range(2):
                    scatter(k, g, h, p)
            return carry

        lax.fori_loop(0, chunks // 2, outer, 0)

        for g in range(4):
            for p in range(2):
                scatter_wait(g, p)

    return emb_kernel


def kernel(input_ids, token_embeddings, position_embeddings):
    B, S = input_ids.shape
    V, D = token_embeddings.shape
    ids = input_ids.reshape(-1).astype(jnp.int32)
    k = _make_kernel(B, S, V, D, 16)
    out = k(ids, token_embeddings, position_embeddings)
    return out.reshape(B, S, D)
